# Initial kernel scaffold; baseline (speedup 1.0000x reference)
import functools

import jax
import jax.numpy as jnp
from jax.experimental import pallas as pl
from jax.experimental.pallas import tpu as pltpu

_E = 8
_B = 2
_DIMS = [80, 64, 48, 32]
_OUTS = [64, 48, 32, 16]
_HS = [16, 32, 64, 128]


def _route_body(txt_ref, wg_ref, idx_ref, wts_ref, mi_ref):
    logits = jnp.dot(txt_ref[:], wg_ref[:], preferred_element_type=jnp.float32)
    iota = jax.lax.broadcasted_iota(jnp.int32, (_B, _E), 1)
    mi_total = jnp.float32(0.0)
    for k in range(4):
        lg = logits[:, k * _E:(k + 1) * _E]
        m = jnp.max(lg, axis=-1, keepdims=True)
        ex = jnp.exp(lg - m)
        probs = ex / jnp.sum(ex, axis=-1, keepdims=True)
        m1 = jnp.max(probs, axis=-1, keepdims=True)
        i1 = jnp.min(jnp.where(probs == m1, iota, _E), axis=-1, keepdims=True)
        pm = jnp.where(iota == i1, -jnp.inf, probs)
        m2 = jnp.max(pm, axis=-1, keepdims=True)
        i2 = jnp.min(jnp.where(pm == m2, iota, _E), axis=-1, keepdims=True)
        s = m1 + m2
        idx_ref[k] = jnp.concatenate([i1, i2], axis=1)
        wts_ref[k] = jnp.concatenate([m1 / s, m2 / s], axis=1)
        importance = jnp.mean(probs, axis=0, keepdims=True)
        sel = (iota == i1) | (iota == i2)
        load = jnp.mean(sel.astype(jnp.float32), axis=0, keepdims=True)
        mi_total = mi_total + _E * jnp.sum(importance * load)
    mi_ref[:] = jnp.full((1, 1), mi_total, jnp.float32)


def _stage_body(blk, d, o, H, W, has_skip, final, *refs):
    if has_skip:
        (idx_ref, wts_ref, x_ref, skip_ref, w1_ref, w2_ref, wu_ref), rest = refs[:7], refs[7:]
    else:
        (idx_ref, wts_ref, x_ref, w1_ref, w2_ref, wu_ref), rest = refs[:6], refs[6:]
        skip_ref = None
    if final:
        wz_ref, out_ref, xpad_ref, zpad_ref = rest
    else:
        out_ref, xpad_ref = rest
        wz_ref = zpad_ref = None

    T = _B * H * W
    Tb = H * W
    pad = W + 8
    logw = W.bit_length() - 1

    x = x_ref[:]
    if skip_ref is not None:
        x = x + skip_ref[:]

    for b in range(_B):
        i1 = idx_ref[blk, 2 * b + 0]
        i2 = idx_ref[blk, 2 * b + 1]
        g1 = wts_ref[blk, 2 * b + 0]
        g2 = wts_ref[blk, 2 * b + 1]
        xb = x[b * Tb:(b + 1) * Tb]
        w1a = w1_ref[pl.ds(i1, 1)][0]
        w1b = w1_ref[pl.ds(i2, 1)][0]
        h1 = jax.nn.gelu(jnp.dot(xb, w1a, preferred_element_type=jnp.float32))
        h2 = jax.nn.gelu(jnp.dot(xb, w1b, preferred_element_type=jnp.float32))
        w2a = w2_ref[pl.ds(i1, 1)][0]
        w2b = w2_ref[pl.ds(i2, 1)][0]
        yb = xb + g1 * jnp.dot(h1, w2a, preferred_element_type=jnp.float32) \
                + g2 * jnp.dot(h2, w2b, preferred_element_type=jnp.float32)
        xpad_ref[pad + b * Tb: pad + (b + 1) * Tb] = yb

    ii = jax.lax.broadcasted_iota(jnp.int32, (T, 1), 0)
    wcol = ii & (W - 1)
    hrow = (ii >> logw) & (H - 1)
    acc = None
    valids = {}
    for dh in (-1, 0, 1):
        for dw in (-1, 0, 1):
            off = dh * W + dw
            xs = xpad_ref[pad + off: pad + off + T]
            valid = ((wcol >= -dw) & (wcol < W - dw)
                     & (hrow >= -dh) & (hrow < H - dh))
            valids[(dh, dw)] = valid
            tap = 3 * (dh + 1) + (dw + 1)
            c = jnp.dot(jnp.where(valid, xs, 0.0), wu_ref[tap],
                        preferred_element_type=jnp.float32)
            acc = c if acc is None else acc + c
    y = jnp.where(acc >= 0, acc, 0.01 * acc)

    if not final:
        out_ref[:] = y
        return

    z = jnp.dot(y, wz_ref[:], preferred_element_type=jnp.float32)
    zpad_ref[pad: pad + T] = z
    img = None
    for dh in (-1, 0, 1):
        for dw in (-1, 0, 1):
            off = dh * W + dw
            tap = 3 * (dh + 1) + (dw + 1)
            zs = zpad_ref[pad + off: pad + off + T, 4 * tap: 4 * tap + 4]
            c = jnp.where(valids[(dh, dw)], zs, 0.0)
            img = c if img is None else img + c
    out_ref[:] = img


def _tokens(x):
    b, c, h, w = x.shape
    return x.transpose(0, 2, 3, 1).reshape(b * h * w, c)


def _shuffle_tokens(y, H, W, o):
    y = y.reshape(_B, H, W, o, 2, 2)
    y = y.transpose(0, 1, 4, 2, 5, 3)
    return y.reshape(_B * 2 * H * 2 * W, o)


def _build_wz(wc):
    wz = jnp.zeros((64, 36), jnp.float32)
    for r1 in (0, 1):
        for r2 in (0, 1):
            for sh in (-1, 0, 1):
                for sw in (-1, 0, 1):
                    for s1 in (0, 1):
                        for s2 in (0, 1):
                            dh = 2 * sh + s1 - r1
                            dw = 2 * sw + s2 - r2
                            if -1 <= dh <= 1 and -1 <= dw <= 1:
                                tap = 3 * (sh + 1) + (sw + 1)
                                col = 4 * tap + r1 * 2 + r2
                                wz = wz.at[(s1 * 2 + s2)::4, col].set(
                                    wc[0, :, dh + 1, dw + 1])
    return wz


def _stage_call(blk, x_tok, skip_tok, idx, wts, p, wz=None):
    d = _DIMS[blk]
    o = _OUTS[blk]
    H = W = _HS[blk]
    T = _B * H * W
    pad = W + 8
    final = wz is not None

    wu = p['Wu'].transpose(2, 3, 1, 0).reshape(9, d, 4 * o)
    out_dim = 4 if final else 4 * o
    smem = pl.BlockSpec(memory_space=pltpu.SMEM)
    vmem = pl.BlockSpec(memory_space=pltpu.VMEM)

    args = [idx, wts, x_tok]
    specs = [smem, smem, vmem]
    if skip_tok is not None:
        args.append(skip_tok)
        specs.append(vmem)
    args += [p['W1'], p['W2'], wu]
    specs += [vmem, vmem, vmem]
    scratch = [pltpu.VMEM((T + 2 * pad, d), jnp.float32)]
    if final:
        args.append(wz)
        specs.append(vmem)
        scratch.append(pltpu.VMEM((T + 2 * pad, 36), jnp.float32))

    body = functools.partial(_stage_body, blk, d, o, H, W,
                             skip_tok is not None, final)
    return pl.pallas_call(
        body,
        out_shape=jax.ShapeDtypeStruct((T, out_dim), jnp.float32),
        in_specs=specs,
        out_specs=vmem,
        scratch_shapes=scratch,
    )(*args)


def kernel(x_encode_0, x_encode_1, x_encode_2, x_encode_3, text_feature, params):
    wg = jnp.concatenate([params['blk%d' % k]['Wg'] for k in range(4)], axis=1)
    idx, wts, mi = pl.pallas_call(
        _route_body,
        out_shape=(jax.ShapeDtypeStruct((4, _B, 2), jnp.int32),
                   jax.ShapeDtypeStruct((4, _B, 2), jnp.float32),
                   jax.ShapeDtypeStruct((1, 1), jnp.float32)),
    )(text_feature, wg)
    idx = idx.reshape(4, 2 * _B)
    wts = wts.reshape(4, 2 * _B)

    skips = [None, _tokens(x_encode_2), _tokens(x_encode_1), _tokens(x_encode_0)]
    cur = _tokens(x_encode_3)
    for k in range(3):
        y = _stage_call(k, cur, skips[k], idx, wts, params['blk%d' % k])
        cur = _shuffle_tokens(y, _HS[k], _HS[k], _OUTS[k])

    wz = _build_wz(params['Wc'])
    img_tok = _stage_call(3, cur, skips[3], idx, wts, params['blk3'], wz=wz)

    img = img_tok.reshape(_B, 128, 128, 2, 2)
    img = img.transpose(0, 1, 3, 2, 4).reshape(_B, 1, 256, 256)
    return img, mi.reshape(())


# speedup vs baseline: 1.5162x; 1.5162x over previous
import functools

import jax
import jax.numpy as jnp
from jax.experimental import pallas as pl
from jax.experimental.pallas import tpu as pltpu

_E = 8
_B = 2
_DIMS = [80, 64, 48, 32]
_OUTS = [64, 48, 32, 16]
_HS = [16, 32, 64, 128]
_FFN_S = [1, 1, 2, 4]
_CONV_C = [512, 2048, 4096, 4096]


def _route_body(txt_ref, wg_ref, idx_ref, wts_ref, mi_ref):
    logits = jnp.dot(txt_ref[:], wg_ref[:], preferred_element_type=jnp.float32)
    iota = jax.lax.broadcasted_iota(jnp.int32, (_B, _E), 1)
    mi_total = jnp.float32(0.0)
    for k in range(4):
        lg = logits[:, k * _E:(k + 1) * _E]
        m = jnp.max(lg, axis=-1, keepdims=True)
        ex = jnp.exp(lg - m)
        probs = ex / jnp.sum(ex, axis=-1, keepdims=True)
        m1 = jnp.max(probs, axis=-1, keepdims=True)
        i1 = jnp.min(jnp.where(probs == m1, iota, _E), axis=-1, keepdims=True)
        pm = jnp.where(iota == i1, -jnp.inf, probs)
        m2 = jnp.max(pm, axis=-1, keepdims=True)
        i2 = jnp.min(jnp.where(pm == m2, iota, _E), axis=-1, keepdims=True)
        s = m1 + m2
        idx_ref[k] = jnp.concatenate([i1, i2], axis=1)
        wts_ref[k] = jnp.concatenate([m1 / s, m2 / s], axis=1)
        importance = jnp.mean(probs, axis=0, keepdims=True)
        sel = (iota == i1) | (iota == i2)
        load = jnp.mean(sel.astype(jnp.float32), axis=0, keepdims=True)
        mi_total = mi_total + _E * jnp.sum(importance * load)
    mi_ref[:] = jnp.full((1, 1), mi_total, jnp.float32)


def _ffn_body(blk, has_skip, *refs):
    if has_skip:
        idx_ref, wts_ref, x_ref, skip_ref, w1_ref, w2_ref, out_ref = refs
    else:
        idx_ref, wts_ref, x_ref, w1_ref, w2_ref, out_ref = refs
        skip_ref = None
    b = pl.program_id(0)
    x = x_ref[:]
    if skip_ref is not None:
        x = x + skip_ref[:]
    i1 = idx_ref[blk, 2 * b + 0]
    i2 = idx_ref[blk, 2 * b + 1]
    g1 = wts_ref[blk, 2 * b + 0]
    g2 = wts_ref[blk, 2 * b + 1]
    w1a = w1_ref[pl.ds(i1, 1)][0]
    w1b = w1_ref[pl.ds(i2, 1)][0]
    h1 = jax.nn.gelu(jnp.dot(x, w1a, preferred_element_type=jnp.float32))
    h2 = jax.nn.gelu(jnp.dot(x, w1b, preferred_element_type=jnp.float32))
    w2a = w2_ref[pl.ds(i1, 1)][0]
    w2b = w2_ref[pl.ds(i2, 1)][0]
    out_ref[:] = x + g1 * jnp.dot(h1, w2a, preferred_element_type=jnp.float32) \
                   + g2 * jnp.dot(h2, w2b, preferred_element_type=jnp.float32)


def _conv_body(d, H, W, C, with_z, *refs):
    if with_z:
        ym1_ref, y0_ref, yp1_ref, wu_ref, wz_ref, out_ref, buf_ref = refs
    else:
        ym1_ref, y0_ref, yp1_ref, wu_ref, out_ref, buf_ref = refs
        wz_ref = None
    i = pl.program_id(0)
    logw = W.bit_length() - 1
    buf_ref[0:C] = ym1_ref[:]
    buf_ref[C:2 * C] = y0_ref[:]
    buf_ref[2 * C:3 * C] = yp1_ref[:]
    gi = i * C + jax.lax.broadcasted_iota(jnp.int32, (C, 1), 0)
    wcol = gi & (W - 1)
    hrow = (gi >> logw) & (H - 1)
    acc = None
    for dh in (-1, 0, 1):
        for dw in (-1, 0, 1):
            off = dh * W + dw
            xs = buf_ref[C + off: 2 * C + off]
            valid = ((wcol >= -dw) & (wcol < W - dw)
                     & (hrow >= -dh) & (hrow < H - dh))
            tap = 3 * (dh + 1) + (dw + 1)
            c = jnp.dot(jnp.where(valid, xs, 0.0), wu_ref[tap],
                        preferred_element_type=jnp.float32)
            acc = c if acc is None else acc + c
    y = jnp.where(acc >= 0, acc, 0.01 * acc)
    if wz_ref is not None:
        y = jnp.dot(y, wz_ref[:], preferred_element_type=jnp.float32)
    out_ref[:] = y


def _img_body(H, W, C, zm1_ref, z0_ref, zp1_ref, out_ref, buf_ref):
    i = pl.program_id(0)
    logw = W.bit_length() - 1
    buf_ref[0:C] = zm1_ref[:]
    buf_ref[C:2 * C] = z0_ref[:]
    buf_ref[2 * C:3 * C] = zp1_ref[:]
    gi = i * C + jax.lax.broadcasted_iota(jnp.int32, (C, 1), 0)
    wcol = gi & (W - 1)
    hrow = (gi >> logw) & (H - 1)
    img = None
    for dh in (-1, 0, 1):
        for dw in (-1, 0, 1):
            off = dh * W + dw
            tap = 3 * (dh + 1) + (dw + 1)
            zs = buf_ref[C + off: 2 * C + off, 4 * tap: 4 * tap + 4]
            valid = ((wcol >= -dw) & (wcol < W - dw)
                     & (hrow >= -dh) & (hrow < H - dh))
            c = jnp.where(valid, zs, 0.0)
            img = c if img is None else img + c
    out_ref[:] = img


def _tokens(x):
    b, c, h, w = x.shape
    return x.transpose(0, 2, 3, 1).reshape(b * h * w, c)


def _shuffle_tokens(y, H, W, o):
    y = y.reshape(_B, H, W, o, 2, 2)
    y = y.transpose(0, 1, 4, 2, 5, 3)
    return y.reshape(_B * 2 * H * 2 * W, o)


def _build_wz(wc):
    wz = jnp.zeros((64, 36), jnp.float32)
    for r1 in (0, 1):
        for r2 in (0, 1):
            for sh in (-1, 0, 1):
                for sw in (-1, 0, 1):
                    for s1 in (0, 1):
                        for s2 in (0, 1):
                            dh = 2 * sh + s1 - r1
                            dw = 2 * sw + s2 - r2
                            if -1 <= dh <= 1 and -1 <= dw <= 1:
                                tap = 3 * (sh + 1) + (sw + 1)
                                col = 4 * tap + r1 * 2 + r2
                                wz = wz.at[(s1 * 2 + s2)::4, col].set(
                                    wc[0, :, dh + 1, dw + 1])
    return wz


def _ffn_call(blk, x_tok, skip_tok, idx, wts, p):
    d = _DIMS[blk]
    H = _HS[blk]
    Tb = H * H
    S = _FFN_S[blk]
    R = Tb // S
    smem = pl.BlockSpec(memory_space=pltpu.SMEM)
    row_spec = pl.BlockSpec((R, d), lambda b, s: (b * S + s, 0))
    full = lambda a: pl.BlockSpec(a.shape, lambda b, s: (0,) * a.ndim)

    args = [idx, wts, x_tok]
    specs = [smem, smem, row_spec]
    if skip_tok is not None:
        args.append(skip_tok)
        specs.append(row_spec)
    args += [p['W1'], p['W2']]
    specs += [full(p['W1']), full(p['W2'])]
    return pl.pallas_call(
        functools.partial(_ffn_body, blk, skip_tok is not None),
        grid=(_B, S),
        out_shape=jax.ShapeDtypeStruct((_B * Tb, d), jnp.float32),
        in_specs=specs,
        out_specs=row_spec,
    )(*args)


def _conv_call(blk, y_tok, p, wz=None):
    d = _DIMS[blk]
    o = _OUTS[blk]
    H = W = _HS[blk]
    T = _B * H * W
    C = _CONV_C[blk]
    N = T // C
    wu = p['Wu'].transpose(2, 3, 1, 0).reshape(9, d, 4 * o)
    out_dim = 36 if wz is not None else 4 * o

    m1_spec = pl.BlockSpec((C, d), lambda i: (jnp.maximum(i - 1, 0), 0))
    c0_spec = pl.BlockSpec((C, d), lambda i: (i, 0))
    p1_spec = pl.BlockSpec((C, d), lambda i: (jnp.minimum(i + 1, N - 1), 0))
    full = lambda a: pl.BlockSpec(a.shape, lambda i: (0,) * a.ndim)

    args = [y_tok, y_tok, y_tok, wu]
    specs = [m1_spec, c0_spec, p1_spec, full(wu)]
    if wz is not None:
        args.append(wz)
        specs.append(full(wz))
    return pl.pallas_call(
        functools.partial(_conv_body, d, H, W, C, wz is not None),
        grid=(N,),
        out_shape=jax.ShapeDtypeStruct((T, out_dim), jnp.float32),
        in_specs=specs,
        out_specs=pl.BlockSpec((C, out_dim), lambda i: (i, 0)),
        scratch_shapes=[pltpu.VMEM((3 * C, d), jnp.float32)],
    )(*args)


def _img_call(z_tok):
    H = W = 128
    T = _B * H * W
    C = 4096
    N = T // C
    m1_spec = pl.BlockSpec((C, 36), lambda i: (jnp.maximum(i - 1, 0), 0))
    c0_spec = pl.BlockSpec((C, 36), lambda i: (i, 0))
    p1_spec = pl.BlockSpec((C, 36), lambda i: (jnp.minimum(i + 1, N - 1), 0))
    return pl.pallas_call(
        functools.partial(_img_body, H, W, C),
        grid=(N,),
        out_shape=jax.ShapeDtypeStruct((T, 4), jnp.float32),
        in_specs=[m1_spec, c0_spec, p1_spec],
        out_specs=pl.BlockSpec((C, 4), lambda i: (i, 0)),
        scratch_shapes=[pltpu.VMEM((3 * C, 36), jnp.float32)],
    )(z_tok, z_tok, z_tok)


def kernel(x_encode_0, x_encode_1, x_encode_2, x_encode_3, text_feature, params):
    wg = jnp.concatenate([params['blk%d' % k]['Wg'] for k in range(4)], axis=1)
    idx, wts, mi = pl.pallas_call(
        _route_body,
        out_shape=(jax.ShapeDtypeStruct((4, _B, 2), jnp.int32),
                   jax.ShapeDtypeStruct((4, _B, 2), jnp.float32),
                   jax.ShapeDtypeStruct((1, 1), jnp.float32)),
    )(text_feature, wg)
    idx = idx.reshape(4, 2 * _B)
    wts = wts.reshape(4, 2 * _B)

    skips = [None, _tokens(x_encode_2), _tokens(x_encode_1), _tokens(x_encode_0)]
    cur = _tokens(x_encode_3)
    for k in range(3):
        ff = _ffn_call(k, cur, skips[k], idx, wts, params['blk%d' % k])
        y = _conv_call(k, ff, params['blk%d' % k])
        cur = _shuffle_tokens(y, _HS[k], _HS[k], _OUTS[k])

    wz = _build_wz(params['Wc'])
    ff3 = _ffn_call(3, cur, skips[3], idx, wts, params['blk3'])
    z = _conv_call(3, ff3, params['blk3'], wz=wz)
    img_tok = _img_call(z)

    img = img_tok.reshape(_B, 128, 128, 2, 2)
    img = img.transpose(0, 1, 3, 2, 4).reshape(_B, 1, 256, 256)
    return img, mi.reshape(())


# speedup vs baseline: 1.5247x; 1.0057x over previous
import functools

import jax
import jax.numpy as jnp
from jax.experimental import pallas as pl
from jax.experimental.pallas import tpu as pltpu

_E = 8
_B = 2
_DIMS = [80, 64, 48, 32]
_OUTS = [64, 48, 32, 16]
_HS = [16, 32, 64, 128]
_FFN_S = [1, 1, 2, 4]
_CONV_C = [512, 2048, 4096, 4096]


def _route_body(txt_ref, wg_ref, idx_ref, wts_ref, mi_ref):
    logits = jnp.dot(txt_ref[:], wg_ref[:], preferred_element_type=jnp.float32)
    iota = jax.lax.broadcasted_iota(jnp.int32, (_B, _E), 1)
    mi_total = jnp.float32(0.0)
    for k in range(4):
        lg = logits[:, k * _E:(k + 1) * _E]
        m = jnp.max(lg, axis=-1, keepdims=True)
        ex = jnp.exp(lg - m)
        probs = ex / jnp.sum(ex, axis=-1, keepdims=True)
        m1 = jnp.max(probs, axis=-1, keepdims=True)
        i1 = jnp.min(jnp.where(probs == m1, iota, _E), axis=-1, keepdims=True)
        pm = jnp.where(iota == i1, -jnp.inf, probs)
        m2 = jnp.max(pm, axis=-1, keepdims=True)
        i2 = jnp.min(jnp.where(pm == m2, iota, _E), axis=-1, keepdims=True)
        s = m1 + m2
        idx_ref[k] = jnp.concatenate([i1, i2], axis=1)
        wts_ref[k] = jnp.concatenate([m1 / s, m2 / s], axis=1)
        importance = jnp.mean(probs, axis=0, keepdims=True)
        sel = (iota == i1) | (iota == i2)
        load = jnp.mean(sel.astype(jnp.float32), axis=0, keepdims=True)
        mi_total = mi_total + _E * jnp.sum(importance * load)
    mi_ref[:] = jnp.full((1, 1), mi_total, jnp.float32)


def _ffn_body(blk, has_skip, *refs):
    if has_skip:
        idx_ref, wts_ref, x_ref, skip_ref, w1_ref, w2_ref, out_ref = refs
    else:
        idx_ref, wts_ref, x_ref, w1_ref, w2_ref, out_ref = refs
        skip_ref = None
    b = pl.program_id(0)
    x = x_ref[:]
    if skip_ref is not None:
        x = x + skip_ref[:]
    i1 = idx_ref[blk, 2 * b + 0]
    i2 = idx_ref[blk, 2 * b + 1]
    g1 = wts_ref[blk, 2 * b + 0]
    g2 = wts_ref[blk, 2 * b + 1]
    w1a = w1_ref[pl.ds(i1, 1)][0]
    w1b = w1_ref[pl.ds(i2, 1)][0]
    xb = x.astype(jnp.bfloat16)
    h1 = jax.nn.gelu(jnp.dot(xb, w1a, preferred_element_type=jnp.float32))
    h2 = jax.nn.gelu(jnp.dot(xb, w1b, preferred_element_type=jnp.float32))
    w2a = w2_ref[pl.ds(i1, 1)][0]
    w2b = w2_ref[pl.ds(i2, 1)][0]
    out_ref[:] = x + g1 * jnp.dot(h1.astype(jnp.bfloat16), w2a,
                                  preferred_element_type=jnp.float32) \
                   + g2 * jnp.dot(h2.astype(jnp.bfloat16), w2b,
                                  preferred_element_type=jnp.float32)


def _conv_body(d, H, W, C, with_z, *refs):
    if with_z:
        ym1_ref, y0_ref, yp1_ref, wu_ref, wz_ref, out_ref, buf_ref = refs
    else:
        ym1_ref, y0_ref, yp1_ref, wu_ref, out_ref, buf_ref = refs
        wz_ref = None
    i = pl.program_id(0)
    logw = W.bit_length() - 1
    buf_ref[0:C] = ym1_ref[:]
    buf_ref[C:2 * C] = y0_ref[:]
    buf_ref[2 * C:3 * C] = yp1_ref[:]
    gi = i * C + jax.lax.broadcasted_iota(jnp.int32, (C, 1), 0)
    wcol = gi & (W - 1)
    hrow = (gi >> logw) & (H - 1)
    acc = None
    for dh in (-1, 0, 1):
        for dw in (-1, 0, 1):
            off = dh * W + dw
            xs = buf_ref[C + off: 2 * C + off]
            valid = ((wcol >= -dw) & (wcol < W - dw)
                     & (hrow >= -dh) & (hrow < H - dh))
            tap = 3 * (dh + 1) + (dw + 1)
            c = jnp.dot(jnp.where(valid, xs, 0.0).astype(jnp.bfloat16),
                        wu_ref[tap], preferred_element_type=jnp.float32)
            acc = c if acc is None else acc + c
    y = jnp.where(acc >= 0, acc, 0.01 * acc)
    if wz_ref is not None:
        y = jnp.dot(y.astype(jnp.bfloat16), wz_ref[:],
                    preferred_element_type=jnp.float32)
    out_ref[:] = y


def _img_body(H, W, C, zm1_ref, z0_ref, zp1_ref, out_ref, buf_ref):
    i = pl.program_id(0)
    logw = W.bit_length() - 1
    buf_ref[0:C] = zm1_ref[:]
    buf_ref[C:2 * C] = z0_ref[:]
    buf_ref[2 * C:3 * C] = zp1_ref[:]
    gi = i * C + jax.lax.broadcasted_iota(jnp.int32, (C, 1), 0)
    wcol = gi & (W - 1)
    hrow = (gi >> logw) & (H - 1)
    img = None
    for dh in (-1, 0, 1):
        for dw in (-1, 0, 1):
            off = dh * W + dw
            tap = 3 * (dh + 1) + (dw + 1)
            zs = buf_ref[C + off: 2 * C + off, 4 * tap: 4 * tap + 4]
            valid = ((wcol >= -dw) & (wcol < W - dw)
                     & (hrow >= -dh) & (hrow < H - dh))
            c = jnp.where(valid, zs, 0.0)
            img = c if img is None else img + c
    out_ref[:] = img


def _tokens(x):
    b, c, h, w = x.shape
    return x.transpose(0, 2, 3, 1).reshape(b * h * w, c)


def _shuffle_tokens(y, H, W, o):
    y = y.reshape(_B, H, W, o, 2, 2)
    y = y.transpose(0, 1, 4, 2, 5, 3)
    return y.reshape(_B * 2 * H * 2 * W, o)


def _build_wz(wc):
    wz = jnp.zeros((64, 36), jnp.float32)
    for r1 in (0, 1):
        for r2 in (0, 1):
            for sh in (-1, 0, 1):
                for sw in (-1, 0, 1):
                    for s1 in (0, 1):
                        for s2 in (0, 1):
                            dh = 2 * sh + s1 - r1
                            dw = 2 * sw + s2 - r2
                            if -1 <= dh <= 1 and -1 <= dw <= 1:
                                tap = 3 * (sh + 1) + (sw + 1)
                                col = 4 * tap + r1 * 2 + r2
                                wz = wz.at[(s1 * 2 + s2)::4, col].set(
                                    wc[0, :, dh + 1, dw + 1])
    return wz


def _ffn_call(blk, x_tok, skip_tok, idx, wts, p):
    d = _DIMS[blk]
    H = _HS[blk]
    Tb = H * H
    S = _FFN_S[blk]
    R = Tb // S
    smem = pl.BlockSpec(memory_space=pltpu.SMEM)
    row_spec = pl.BlockSpec((R, d), lambda b, s: (b * S + s, 0))
    full = lambda a: pl.BlockSpec(a.shape, lambda b, s: (0,) * a.ndim)

    args = [idx, wts, x_tok]
    specs = [smem, smem, row_spec]
    if skip_tok is not None:
        args.append(skip_tok)
        specs.append(row_spec)
    w1 = p['W1'].astype(jnp.bfloat16)
    w2 = p['W2'].astype(jnp.bfloat16)
    args += [w1, w2]
    specs += [full(w1), full(w2)]
    return pl.pallas_call(
        functools.partial(_ffn_body, blk, skip_tok is not None),
        grid=(_B, S),
        out_shape=jax.ShapeDtypeStruct((_B * Tb, d), jnp.float32),
        in_specs=specs,
        out_specs=row_spec,
    )(*args)


def _conv_call(blk, y_tok, p, wz=None):
    d = _DIMS[blk]
    o = _OUTS[blk]
    H = W = _HS[blk]
    T = _B * H * W
    C = _CONV_C[blk]
    N = T // C
    wu = p['Wu'].transpose(2, 3, 1, 0).reshape(9, d, 4 * o).astype(jnp.bfloat16)
    out_dim = 36 if wz is not None else 4 * o

    m1_spec = pl.BlockSpec((C, d), lambda i: (jnp.maximum(i - 1, 0), 0))
    c0_spec = pl.BlockSpec((C, d), lambda i: (i, 0))
    p1_spec = pl.BlockSpec((C, d), lambda i: (jnp.minimum(i + 1, N - 1), 0))
    full = lambda a: pl.BlockSpec(a.shape, lambda i: (0,) * a.ndim)

    args = [y_tok, y_tok, y_tok, wu]
    specs = [m1_spec, c0_spec, p1_spec, full(wu)]
    if wz is not None:
        wzb = wz.astype(jnp.bfloat16)
        args.append(wzb)
        specs.append(full(wzb))
    return pl.pallas_call(
        functools.partial(_conv_body, d, H, W, C, wz is not None),
        grid=(N,),
        out_shape=jax.ShapeDtypeStruct((T, out_dim), jnp.float32),
        in_specs=specs,
        out_specs=pl.BlockSpec((C, out_dim), lambda i: (i, 0)),
        scratch_shapes=[pltpu.VMEM((3 * C, d), jnp.float32)],
    )(*args)


def _img_call(z_tok):
    H = W = 128
    T = _B * H * W
    C = 4096
    N = T // C
    m1_spec = pl.BlockSpec((C, 36), lambda i: (jnp.maximum(i - 1, 0), 0))
    c0_spec = pl.BlockSpec((C, 36), lambda i: (i, 0))
    p1_spec = pl.BlockSpec((C, 36), lambda i: (jnp.minimum(i + 1, N - 1), 0))
    return pl.pallas_call(
        functools.partial(_img_body, H, W, C),
        grid=(N,),
        out_shape=jax.ShapeDtypeStruct((T, 4), jnp.float32),
        in_specs=[m1_spec, c0_spec, p1_spec],
        out_specs=pl.BlockSpec((C, 4), lambda i: (i, 0)),
        scratch_shapes=[pltpu.VMEM((3 * C, 36), jnp.float32)],
    )(z_tok, z_tok, z_tok)


def kernel(x_encode_0, x_encode_1, x_encode_2, x_encode_3, text_feature, params):
    wg = jnp.concatenate([params['blk%d' % k]['Wg'] for k in range(4)], axis=1)
    idx, wts, mi = pl.pallas_call(
        _route_body,
        out_shape=(jax.ShapeDtypeStruct((4, _B, 2), jnp.int32),
                   jax.ShapeDtypeStruct((4, _B, 2), jnp.float32),
                   jax.ShapeDtypeStruct((1, 1), jnp.float32)),
    )(text_feature, wg)
    idx = idx.reshape(4, 2 * _B)
    wts = wts.reshape(4, 2 * _B)

    skips = [None, _tokens(x_encode_2), _tokens(x_encode_1), _tokens(x_encode_0)]
    cur = _tokens(x_encode_3)
    for k in range(3):
        ff = _ffn_call(k, cur, skips[k], idx, wts, params['blk%d' % k])
        y = _conv_call(k, ff, params['blk%d' % k])
        cur = _shuffle_tokens(y, _HS[k], _HS[k], _OUTS[k])

    wz = _build_wz(params['Wc'])
    ff3 = _ffn_call(3, cur, skips[3], idx, wts, params['blk3'])
    z = _conv_call(3, ff3, params['blk3'], wz=wz)
    img_tok = _img_call(z)

    img = img_tok.reshape(_B, 128, 128, 2, 2)
    img = img.transpose(0, 1, 3, 2, 4).reshape(_B, 1, 256, 256)
    return img, mi.reshape(())


# speedup vs baseline: 2.3070x; 1.5130x over previous
import functools

import jax
import jax.numpy as jnp
from jax.experimental import pallas as pl
from jax.experimental.pallas import tpu as pltpu

_E = 8
_B = 2
_DIMS = [80, 64, 48, 32]
_OUTS = [64, 48, 32, 16]
_HS = [16, 32, 64, 128]

_CT0 = (((0,), (0,)), ((), ()))


def _route_body(txt_ref, wg_ref, idx_ref, wts_ref, mi_ref):
    logits = jnp.dot(txt_ref[:], wg_ref[:], preferred_element_type=jnp.float32)
    iota = jax.lax.broadcasted_iota(jnp.int32, (_B, _E), 1)
    mi_total = jnp.float32(0.0)
    for k in range(4):
        lg = logits[:, k * _E:(k + 1) * _E]
        m = jnp.max(lg, axis=-1, keepdims=True)
        ex = jnp.exp(lg - m)
        probs = ex / jnp.sum(ex, axis=-1, keepdims=True)
        m1 = jnp.max(probs, axis=-1, keepdims=True)
        i1 = jnp.min(jnp.where(probs == m1, iota, _E), axis=-1, keepdims=True)
        pm = jnp.where(iota == i1, -jnp.inf, probs)
        m2 = jnp.max(pm, axis=-1, keepdims=True)
        i2 = jnp.min(jnp.where(pm == m2, iota, _E), axis=-1, keepdims=True)
        s = m1 + m2
        idx_ref[k] = jnp.concatenate([i1, i2], axis=1)
        wts_ref[k] = jnp.concatenate([m1 / s, m2 / s], axis=1)
        importance = jnp.mean(probs, axis=0, keepdims=True)
        sel = (iota == i1) | (iota == i2)
        load = jnp.mean(sel.astype(jnp.float32), axis=0, keepdims=True)
        mi_total = mi_total + _E * jnp.sum(importance * load)
    mi_ref[:] = jnp.full((1, 1), mi_total, jnp.float32)


def _dotg(a, b):
    return jax.lax.dot_general(a, b.astype(jnp.bfloat16), _CT0,
                               preferred_element_type=jnp.float32)


def _stage_body(blk, final, *refs):
    if final:
        idx_ref, wts_ref, x_ref, w1_ref, w2_ref, wu_ref, wz_ref, out_ref, \
            ybuf_ref, zbuf_ref = refs
    else:
        idx_ref, wts_ref, x_ref, w1_ref, w2_ref, wu_ref, out_ref, ybuf_ref = refs
        wz_ref = zbuf_ref = None

    d = _DIMS[blk]
    H = W = _HS[blk]
    Tb = H * W
    P = W + 8
    logw = W.bit_length() - 1

    jj = jax.lax.broadcasted_iota(jnp.int32, (1, Tb), 1)
    wcol = jj & (W - 1)
    hrow = jj >> logw

    for b in range(_B):
        i1 = idx_ref[blk, 2 * b + 0]
        i2 = idx_ref[blk, 2 * b + 1]
        g1 = wts_ref[blk, 2 * b + 0]
        g2 = wts_ref[blk, 2 * b + 1]
        x = x_ref[b]
        xb = x.astype(jnp.bfloat16)
        w1a = w1_ref[pl.ds(i1, 1)][0]
        w1b = w1_ref[pl.ds(i2, 1)][0]
        h1 = jax.nn.gelu(_dotg(w1a, xb))
        h2 = jax.nn.gelu(_dotg(w1b, xb))
        w2a = w2_ref[pl.ds(i1, 1)][0]
        w2b = w2_ref[pl.ds(i2, 1)][0]
        y = x + g1 * _dotg(w2a, h1) + g2 * _dotg(w2b, h2)
        ybuf_ref[:, P:P + Tb] = y

        acc = None
        for dh in (-1, 0, 1):
            for dw in (-1, 0, 1):
                off = dh * W + dw
                ys = ybuf_ref[:, P + off:P + off + Tb]
                valid = ((wcol >= -dw) & (wcol < W - dw)
                         & (hrow >= -dh) & (hrow < H - dh))
                tap = 3 * (dh + 1) + (dw + 1)
                c = _dotg(wu_ref[tap], jnp.where(valid, ys, 0.0))
                acc = c if acc is None else acc + c
        y2 = jnp.where(acc >= 0, acc, 0.01 * acc)

        if not final:
            out_ref[b] = y2
            continue

        z = _dotg(wz_ref[:], y2)
        zbuf_ref[:, P:P + Tb] = z
        img = None
        for dh in (-1, 0, 1):
            for dw in (-1, 0, 1):
                off = dh * W + dw
                tap = 3 * (dh + 1) + (dw + 1)
                zs = zbuf_ref[4 * tap:4 * tap + 4, P + off:P + off + Tb]
                valid = ((wcol >= -dw) & (wcol < W - dw)
                         & (hrow >= -dh) & (hrow < H - dh))
                c = jnp.where(valid, zs, 0.0)
                img = c if img is None else img + c
        out_ref[b] = img


def _build_wz(wc):
    wz = jnp.zeros((64, 36), jnp.float32)
    for r1 in (0, 1):
        for r2 in (0, 1):
            for sh in (-1, 0, 1):
                for sw in (-1, 0, 1):
                    for s1 in (0, 1):
                        for s2 in (0, 1):
                            dh = 2 * sh + s1 - r1
                            dw = 2 * sw + s2 - r2
                            if -1 <= dh <= 1 and -1 <= dw <= 1:
                                tap = 3 * (sh + 1) + (sw + 1)
                                col = 4 * tap + r1 * 2 + r2
                                wz = wz.at[(s1 * 2 + s2)::4, col].set(
                                    wc[0, :, dh + 1, dw + 1])
    return wz


def _shuffle(y, o):
    n, c4, h, w = y.shape
    y = y.reshape(n, o, 2, 2, h, w)
    y = y.transpose(0, 1, 4, 2, 5, 3)
    return y.reshape(n, o, 2 * h, 2 * w)


def _stage_call(blk, x, idx, wts, p, wz=None):
    d = _DIMS[blk]
    o = _OUTS[blk]
    H = _HS[blk]
    Tb = H * H
    P = H + 8
    final = wz is not None

    w1 = p['W1'].astype(jnp.bfloat16)
    w2 = p['W2'].astype(jnp.bfloat16)
    wu = p['Wu'].transpose(2, 3, 1, 0).reshape(9, d, 4 * o).astype(jnp.bfloat16)
    smem = pl.BlockSpec(memory_space=pltpu.SMEM)
    vmem = pl.BlockSpec(memory_space=pltpu.VMEM)

    args = [idx, wts, x.reshape(_B, d, Tb), w1, w2, wu]
    specs = [smem, smem, vmem, vmem, vmem, vmem]
    scratch = [pltpu.VMEM((d, Tb + 2 * P), jnp.float32)]
    out_c = 4 if final else 4 * o
    if final:
        args.append(wz.astype(jnp.bfloat16))
        specs.append(vmem)
        scratch.append(pltpu.VMEM((36, Tb + 2 * P), jnp.float32))

    y = pl.pallas_call(
        functools.partial(_stage_body, blk, final),
        out_shape=jax.ShapeDtypeStruct((_B, out_c, Tb), jnp.float32),
        in_specs=specs,
        out_specs=vmem,
        scratch_shapes=scratch,
    )(*args)
    return y.reshape(_B, out_c, H, H)


def kernel(x_encode_0, x_encode_1, x_encode_2, x_encode_3, text_feature, params):
    wg = jnp.concatenate([params['blk%d' % k]['Wg'] for k in range(4)], axis=1)
    idx, wts, mi = pl.pallas_call(
        _route_body,
        out_shape=(jax.ShapeDtypeStruct((4, _B, 2), jnp.int32),
                   jax.ShapeDtypeStruct((4, _B, 2), jnp.float32),
                   jax.ShapeDtypeStruct((1, 1), jnp.float32)),
    )(text_feature, wg)
    idx = idx.reshape(4, 2 * _B)
    wts = wts.reshape(4, 2 * _B)

    skips = [None, x_encode_2, x_encode_1, x_encode_0]
    cur = x_encode_3
    for k in range(3):
        y = _stage_call(k, cur, idx, wts, params['blk%d' % k])
        cur = skips[k + 1] + _shuffle(y, _OUTS[k])

    wz = _build_wz(params['Wc'])
    img4 = _stage_call(3, cur, idx, wts, params['blk3'], wz=wz)
    img = _shuffle(img4, 1)
    return img, mi.reshape(())


# speedup vs baseline: 2.3474x; 1.0175x over previous
import functools

import jax
import jax.numpy as jnp
from jax.experimental import pallas as pl
from jax.experimental.pallas import tpu as pltpu

_E = 8
_B = 2
_DIMS = [80, 64, 48, 32]
_OUTS = [64, 48, 32, 16]
_HS = [16, 32, 64, 128]

_CT0 = (((0,), (0,)), ((), ()))


def _route_body(txt_ref, wg_ref, idx_ref, wts_ref, mi_ref):
    logits = jnp.dot(txt_ref[:], wg_ref[:], preferred_element_type=jnp.float32)
    iota = jax.lax.broadcasted_iota(jnp.int32, (_B, _E), 1)
    mi_total = jnp.float32(0.0)
    for k in range(4):
        lg = logits[:, k * _E:(k + 1) * _E]
        m = jnp.max(lg, axis=-1, keepdims=True)
        ex = jnp.exp(lg - m)
        probs = ex / jnp.sum(ex, axis=-1, keepdims=True)
        m1 = jnp.max(probs, axis=-1, keepdims=True)
        i1 = jnp.min(jnp.where(probs == m1, iota, _E), axis=-1, keepdims=True)
        pm = jnp.where(iota == i1, -jnp.inf, probs)
        m2 = jnp.max(pm, axis=-1, keepdims=True)
        i2 = jnp.min(jnp.where(pm == m2, iota, _E), axis=-1, keepdims=True)
        s = m1 + m2
        idx_ref[k] = jnp.concatenate([i1, i2], axis=1)
        wts_ref[k] = jnp.concatenate([m1 / s, m2 / s], axis=1)
        importance = jnp.mean(probs, axis=0, keepdims=True)
        sel = (iota == i1) | (iota == i2)
        load = jnp.mean(sel.astype(jnp.float32), axis=0, keepdims=True)
        mi_total = mi_total + _E * jnp.sum(importance * load)
    mi_ref[:] = jnp.full((1, 1), mi_total, jnp.float32)


def _dotg(a, b):
    return jax.lax.dot_general(a, b.astype(jnp.bfloat16), _CT0,
                               preferred_element_type=jnp.float32)


def _stage_body(blk, final, *refs):
    if final:
        idx_ref, wts_ref, x_ref, w1_ref, w2_ref, wu_ref, wz_ref, out_ref, \
            ybuf_ref, zbuf_ref = refs
    else:
        idx_ref, wts_ref, x_ref, w1_ref, w2_ref, wu_ref, out_ref, ybuf_ref = refs
        wz_ref = zbuf_ref = None

    d = _DIMS[blk]
    H = W = _HS[blk]
    Tb = H * W
    P = W + 8
    logw = W.bit_length() - 1

    jj = jax.lax.broadcasted_iota(jnp.int32, (1, Tb), 1)
    wcol = jj & (W - 1)
    hrow = jj >> logw
    wmask = {-1: wcol >= 1, 0: None, 1: wcol < W - 1}

    ybuf_ref[:, 0:P] = jnp.zeros((d, P), jnp.float32)
    ybuf_ref[:, P + Tb:2 * P + Tb] = jnp.zeros((d, P), jnp.float32)
    if final:
        zbuf_ref[:, 0:P] = jnp.zeros((36, P), jnp.float32)
        zbuf_ref[:, P + Tb:2 * P + Tb] = jnp.zeros((36, P), jnp.float32)

    for b in range(_B):
        i1 = idx_ref[blk, 2 * b + 0]
        i2 = idx_ref[blk, 2 * b + 1]
        g1 = wts_ref[blk, 2 * b + 0]
        g2 = wts_ref[blk, 2 * b + 1]
        x = x_ref[b]
        xb = x.astype(jnp.bfloat16)
        w1a = w1_ref[pl.ds(i1, 1)][0]
        w1b = w1_ref[pl.ds(i2, 1)][0]
        h1 = jax.nn.gelu(_dotg(w1a, xb))
        h2 = jax.nn.gelu(_dotg(w1b, xb))
        w2a = w2_ref[pl.ds(i1, 1)][0]
        w2b = w2_ref[pl.ds(i2, 1)][0]
        y = x + g1 * _dotg(w2a, h1) + g2 * _dotg(w2b, h2)
        ybuf_ref[:, P:P + Tb] = y

        acc = None
        for dw in (-1, 0, 1):
            part = None
            for dh in (-1, 0, 1):
                off = dh * W + dw
                ys = ybuf_ref[:, P + off:P + off + Tb]
                tap = 3 * (dh + 1) + (dw + 1)
                c = _dotg(wu_ref[tap], ys)
                part = c if part is None else part + c
            if wmask[dw] is not None:
                part = jnp.where(wmask[dw], part, 0.0)
            acc = part if acc is None else acc + part
        y2 = jnp.where(acc >= 0, acc, 0.01 * acc)

        if not final:
            out_ref[b] = y2
            continue

        z = _dotg(wz_ref[:], y2)
        zbuf_ref[:, P:P + Tb] = z
        img = None
        for dw in (-1, 0, 1):
            part = None
            for dh in (-1, 0, 1):
                off = dh * W + dw
                tap = 3 * (dh + 1) + (dw + 1)
                zs = zbuf_ref[4 * tap:4 * tap + 4, P + off:P + off + Tb]
                part = zs if part is None else part + zs
            if wmask[dw] is not None:
                part = jnp.where(wmask[dw], part, 0.0)
            img = part if img is None else img + part
        out_ref[b] = img


def _build_wz(wc):
    wz = jnp.zeros((64, 36), jnp.float32)
    for r1 in (0, 1):
        for r2 in (0, 1):
            for sh in (-1, 0, 1):
                for sw in (-1, 0, 1):
                    for s1 in (0, 1):
                        for s2 in (0, 1):
                            dh = 2 * sh + s1 - r1
                            dw = 2 * sw + s2 - r2
                            if -1 <= dh <= 1 and -1 <= dw <= 1:
                                tap = 3 * (sh + 1) + (sw + 1)
                                col = 4 * tap + r1 * 2 + r2
                                wz = wz.at[(s1 * 2 + s2)::4, col].set(
                                    wc[0, :, dh + 1, dw + 1])
    return wz


def _shuffle(y, o):
    n, c4, h, w = y.shape
    y = y.reshape(n, o, 2, 2, h, w)
    y = y.transpose(0, 1, 4, 2, 5, 3)
    return y.reshape(n, o, 2 * h, 2 * w)


def _stage_call(blk, x, idx, wts, p, wz=None):
    d = _DIMS[blk]
    o = _OUTS[blk]
    H = _HS[blk]
    Tb = H * H
    P = H + 8
    final = wz is not None

    w1 = p['W1'].astype(jnp.bfloat16)
    w2 = p['W2'].astype(jnp.bfloat16)
    wu = p['Wu'].transpose(2, 3, 1, 0).reshape(9, d, 4 * o).astype(jnp.bfloat16)
    smem = pl.BlockSpec(memory_space=pltpu.SMEM)
    vmem = pl.BlockSpec(memory_space=pltpu.VMEM)

    args = [idx, wts, x.reshape(_B, d, Tb), w1, w2, wu]
    specs = [smem, smem, vmem, vmem, vmem, vmem]
    scratch = [pltpu.VMEM((d, Tb + 2 * P), jnp.float32)]
    out_c = 4 if final else 4 * o
    if final:
        args.append(wz.astype(jnp.bfloat16))
        specs.append(vmem)
        scratch.append(pltpu.VMEM((36, Tb + 2 * P), jnp.float32))

    y = pl.pallas_call(
        functools.partial(_stage_body, blk, final),
        out_shape=jax.ShapeDtypeStruct((_B, out_c, Tb), jnp.float32),
        in_specs=specs,
        out_specs=vmem,
        scratch_shapes=scratch,
    )(*args)
    return y.reshape(_B, out_c, H, H)


def kernel(x_encode_0, x_encode_1, x_encode_2, x_encode_3, text_feature, params):
    wg = jnp.concatenate([params['blk%d' % k]['Wg'] for k in range(4)], axis=1)
    idx, wts, mi = pl.pallas_call(
        _route_body,
        out_shape=(jax.ShapeDtypeStruct((4, _B, 2), jnp.int32),
                   jax.ShapeDtypeStruct((4, _B, 2), jnp.float32),
                   jax.ShapeDtypeStruct((1, 1), jnp.float32)),
    )(text_feature, wg)
    idx = idx.reshape(4, 2 * _B)
    wts = wts.reshape(4, 2 * _B)

    skips = [None, x_encode_2, x_encode_1, x_encode_0]
    cur = x_encode_3
    for k in range(3):
        y = _stage_call(k, cur, idx, wts, params['blk%d' % k])
        cur = skips[k + 1] + _shuffle(y, _OUTS[k])

    wz = _build_wz(params['Wc'])
    img4 = _stage_call(3, cur, idx, wts, params['blk3'], wz=wz)
    img = _shuffle(img4, 1)
    return img, mi.reshape(())


# speedup vs baseline: 2.6688x; 1.1369x over previous
import functools

import jax
import jax.numpy as jnp
import numpy as np
from jax.experimental import pallas as pl
from jax.experimental.pallas import tpu as pltpu

_E = 8
_B = 2
_DIMS = [80, 64, 48, 32]
_OUTS = [64, 48, 32, 16]
_HS = [16, 32, 64, 128]

_CT0 = (((0,), (0,)), ((), ()))


def _route_body(txt_ref, wg_ref, idx_ref, wts_ref, mi_ref):
    logits = jnp.dot(txt_ref[:], wg_ref[:], preferred_element_type=jnp.float32)
    iota = jax.lax.broadcasted_iota(jnp.int32, (_B, _E), 1)
    mi_total = jnp.float32(0.0)
    for k in range(4):
        lg = logits[:, k * _E:(k + 1) * _E]
        m = jnp.max(lg, axis=-1, keepdims=True)
        ex = jnp.exp(lg - m)
        probs = ex / jnp.sum(ex, axis=-1, keepdims=True)
        m1 = jnp.max(probs, axis=-1, keepdims=True)
        i1 = jnp.min(jnp.where(probs == m1, iota, _E), axis=-1, keepdims=True)
        pm = jnp.where(iota == i1, -jnp.inf, probs)
        m2 = jnp.max(pm, axis=-1, keepdims=True)
        i2 = jnp.min(jnp.where(pm == m2, iota, _E), axis=-1, keepdims=True)
        s = m1 + m2
        idx_ref[k] = jnp.concatenate([i1, i2], axis=1)
        wts_ref[k] = jnp.concatenate([m1 / s, m2 / s], axis=1)
        importance = jnp.mean(probs, axis=0, keepdims=True)
        sel = (iota == i1) | (iota == i2)
        load = jnp.mean(sel.astype(jnp.float32), axis=0, keepdims=True)
        mi_total = mi_total + _E * jnp.sum(importance * load)
    mi_ref[:] = jnp.full((1, 1), mi_total, jnp.float32)


def _dotg(a, b):
    return jax.lax.dot_general(a, b.astype(jnp.bfloat16), _CT0,
                               preferred_element_type=jnp.float32)


def _stage_body(blk, final, *refs):
    if final:
        idx_ref, wts_ref, x_ref, w1_ref, w2_ref, wu_ref, wz_ref, out_ref, \
            ybuf_ref, zbuf_ref = refs
    else:
        idx_ref, wts_ref, x_ref, w1_ref, w2_ref, wu_ref, out_ref, ybuf_ref = refs
        wz_ref = zbuf_ref = None

    d = _DIMS[blk]
    H = W = _HS[blk]
    Tb = H * W
    P = W + 8
    logw = W.bit_length() - 1

    jj = jax.lax.broadcasted_iota(jnp.int32, (1, Tb), 1)
    wcol = jj & (W - 1)
    hrow = jj >> logw
    wmask = {-1: wcol >= 1, 0: None, 1: wcol < W - 1}

    ybuf_ref[:, 0:P] = jnp.zeros((d, P), jnp.float32)
    ybuf_ref[:, P + Tb:2 * P + Tb] = jnp.zeros((d, P), jnp.float32)
    if final:
        zbuf_ref[:, 0:P] = jnp.zeros((36, P), jnp.float32)
        zbuf_ref[:, P + Tb:2 * P + Tb] = jnp.zeros((36, P), jnp.float32)

    for b in range(_B):
        i1 = idx_ref[blk, 2 * b + 0]
        i2 = idx_ref[blk, 2 * b + 1]
        g1 = wts_ref[blk, 2 * b + 0]
        g2 = wts_ref[blk, 2 * b + 1]
        x = x_ref[b]
        xb = x.astype(jnp.bfloat16)
        w1a = w1_ref[pl.ds(i1, 1)][0]
        w1b = w1_ref[pl.ds(i2, 1)][0]
        h1 = jax.nn.gelu(_dotg(w1a, xb))
        h2 = jax.nn.gelu(_dotg(w1b, xb))
        w2a = w2_ref[pl.ds(i1, 1)][0]
        w2b = w2_ref[pl.ds(i2, 1)][0]
        y = x + g1 * _dotg(w2a, h1) + g2 * _dotg(w2b, h2)
        ybuf_ref[:, P:P + Tb] = y

        acc = None
        for dw in (-1, 0, 1):
            part = None
            for dh in (-1, 0, 1):
                off = dh * W + dw
                ys = ybuf_ref[:, P + off:P + off + Tb]
                tap = 3 * (dh + 1) + (dw + 1)
                c = _dotg(wu_ref[tap], ys)
                part = c if part is None else part + c
            if wmask[dw] is not None:
                part = jnp.where(wmask[dw], part, 0.0)
            acc = part if acc is None else acc + part
        y2 = jnp.where(acc >= 0, acc, 0.01 * acc)

        if not final:
            out_ref[b] = y2
            continue

        z = _dotg(wz_ref[:], y2)
        zbuf_ref[:, P:P + Tb] = z
        img = None
        for dw in (-1, 0, 1):
            part = None
            for dh in (-1, 0, 1):
                off = dh * W + dw
                tap = 3 * (dh + 1) + (dw + 1)
                zs = zbuf_ref[4 * tap:4 * tap + 4, P + off:P + off + Tb]
                part = zs if part is None else part + zs
            if wmask[dw] is not None:
                part = jnp.where(wmask[dw], part, 0.0)
            img = part if img is None else img + part
        out_ref[b] = img


def _wz_indices():
    oc = np.zeros((64, 36), np.int32)
    kh = np.zeros((64, 36), np.int32)
    kw = np.zeros((64, 36), np.int32)
    msk = np.zeros((64, 36), np.float32)
    for r1 in (0, 1):
        for r2 in (0, 1):
            for sh in (-1, 0, 1):
                for sw in (-1, 0, 1):
                    for s1 in (0, 1):
                        for s2 in (0, 1):
                            dh = 2 * sh + s1 - r1
                            dw = 2 * sw + s2 - r2
                            if -1 <= dh <= 1 and -1 <= dw <= 1:
                                tap = 3 * (sh + 1) + (sw + 1)
                                col = 4 * tap + r1 * 2 + r2
                                rows = np.arange(s1 * 2 + s2, 64, 4)
                                oc[rows, col] = np.arange(16)
                                kh[rows, col] = dh + 1
                                kw[rows, col] = dw + 1
                                msk[rows, col] = 1.0
    return oc, kh, kw, msk


_WZ_OC, _WZ_KH, _WZ_KW, _WZ_MSK = _wz_indices()


def _build_wz(wc):
    return wc[0][_WZ_OC, _WZ_KH, _WZ_KW] * _WZ_MSK


def _shuffle(y, o):
    n, c4, h, w = y.shape
    y = y.reshape(n, o, 2, 2, h, w)
    y = y.transpose(0, 1, 4, 2, 5, 3)
    return y.reshape(n, o, 2 * h, 2 * w)


def _stage_call(blk, x, idx, wts, p, wz=None):
    d = _DIMS[blk]
    o = _OUTS[blk]
    H = _HS[blk]
    Tb = H * H
    P = H + 8
    final = wz is not None

    w1 = p['W1'].astype(jnp.bfloat16)
    w2 = p['W2'].astype(jnp.bfloat16)
    wu = p['Wu'].transpose(2, 3, 1, 0).reshape(9, d, 4 * o).astype(jnp.bfloat16)
    smem = pl.BlockSpec(memory_space=pltpu.SMEM)
    vmem = pl.BlockSpec(memory_space=pltpu.VMEM)

    args = [idx, wts, x.reshape(_B, d, Tb), w1, w2, wu]
    specs = [smem, smem, vmem, vmem, vmem, vmem]
    scratch = [pltpu.VMEM((d, Tb + 2 * P), jnp.float32)]
    out_c = 4 if final else 4 * o
    if final:
        args.append(wz.astype(jnp.bfloat16))
        specs.append(vmem)
        scratch.append(pltpu.VMEM((36, Tb + 2 * P), jnp.float32))

    y = pl.pallas_call(
        functools.partial(_stage_body, blk, final),
        out_shape=jax.ShapeDtypeStruct((_B, out_c, Tb), jnp.float32),
        in_specs=specs,
        out_specs=vmem,
        scratch_shapes=scratch,
    )(*args)
    return y.reshape(_B, out_c, H, H)


def kernel(x_encode_0, x_encode_1, x_encode_2, x_encode_3, text_feature, params):
    wg = jnp.concatenate([params['blk%d' % k]['Wg'] for k in range(4)], axis=1)
    idx, wts, mi = pl.pallas_call(
        _route_body,
        out_shape=(jax.ShapeDtypeStruct((4, _B, 2), jnp.int32),
                   jax.ShapeDtypeStruct((4, _B, 2), jnp.float32),
                   jax.ShapeDtypeStruct((1, 1), jnp.float32)),
    )(text_feature, wg)
    idx = idx.reshape(4, 2 * _B)
    wts = wts.reshape(4, 2 * _B)

    skips = [None, x_encode_2, x_encode_1, x_encode_0]
    cur = x_encode_3
    for k in range(3):
        y = _stage_call(k, cur, idx, wts, params['blk%d' % k])
        cur = skips[k + 1] + _shuffle(y, _OUTS[k])

    wz = _build_wz(params['Wc'])
    img4 = _stage_call(3, cur, idx, wts, params['blk3'], wz=wz)
    img = _shuffle(img4, 1)
    return img, mi.reshape(())


# speedup vs baseline: 2.6872x; 1.0069x over previous
import functools

import jax
import jax.numpy as jnp
import numpy as np
from jax.experimental import pallas as pl
from jax.experimental.pallas import tpu as pltpu

_E = 8
_B = 2
_DIMS = [80, 64, 48, 32]
_OUTS = [64, 48, 32, 16]
_HS = [16, 32, 64, 128]

_CT0 = (((0,), (0,)), ((), ()))


def _routing_compute(txt_ref, wg_ref, idx_ref, wts_ref, mi_ref):
    logits = jnp.dot(txt_ref[:], wg_ref[:], preferred_element_type=jnp.float32)
    iota = jax.lax.broadcasted_iota(jnp.int32, (_B, _E), 1)
    mi_total = jnp.float32(0.0)
    idx_vecs = []
    wts_vecs = []
    for k in range(4):
        lg = logits[:, k * _E:(k + 1) * _E]
        m = jnp.max(lg, axis=-1, keepdims=True)
        ex = jnp.exp(lg - m)
        probs = ex / jnp.sum(ex, axis=-1, keepdims=True)
        m1 = jnp.max(probs, axis=-1, keepdims=True)
        i1 = jnp.min(jnp.where(probs == m1, iota, _E), axis=-1, keepdims=True)
        pm = jnp.where(iota == i1, -jnp.inf, probs)
        m2 = jnp.max(pm, axis=-1, keepdims=True)
        i2 = jnp.min(jnp.where(pm == m2, iota, _E), axis=-1, keepdims=True)
        s = m1 + m2
        idx_ref[k] = jnp.concatenate([i1, i2], axis=1)
        wts_ref[k] = jnp.concatenate([m1 / s, m2 / s], axis=1)
        importance = jnp.mean(probs, axis=0, keepdims=True)
        sel = (iota == i1) | (iota == i2)
        load = jnp.mean(sel.astype(jnp.float32), axis=0, keepdims=True)
        mi_total = mi_total + _E * jnp.sum(importance * load)
        idx_vecs.append((i1, i2))
        wts_vecs.append((m1 / s, m2 / s))
    mi_ref[:] = jnp.full((1, 1), mi_total, jnp.float32)
    return idx_vecs, wts_vecs


def _dotg(a, b):
    return jax.lax.dot_general(a, b.astype(jnp.bfloat16), _CT0,
                               preferred_element_type=jnp.float32)


def _stage_body(blk, final, *refs):
    route_vecs = None
    if blk == 0:
        txt_ref, wg_ref, x_ref, w1_ref, w2_ref, wu_ref, \
            idx_out, wts_out, mi_out, out_ref, ybuf_ref = refs
        route_vecs = _routing_compute(txt_ref, wg_ref, idx_out, wts_out, mi_out)
        wz_ref = zbuf_ref = None
    elif final:
        idx_ref, wts_ref, x_ref, w1_ref, w2_ref, wu_ref, wz_ref, out_ref, \
            ybuf_ref, zbuf_ref = refs
    else:
        idx_ref, wts_ref, x_ref, w1_ref, w2_ref, wu_ref, out_ref, ybuf_ref = refs
        wz_ref = zbuf_ref = None

    d = _DIMS[blk]
    H = W = _HS[blk]
    Tb = H * W
    P = W + 8
    logw = W.bit_length() - 1

    jj = jax.lax.broadcasted_iota(jnp.int32, (1, Tb), 1)
    wcol = jj & (W - 1)
    hrow = jj >> logw
    wmask = {-1: wcol >= 1, 0: None, 1: wcol < W - 1}

    ybuf_ref[:, 0:P] = jnp.zeros((d, P), jnp.float32)
    ybuf_ref[:, P + Tb:2 * P + Tb] = jnp.zeros((d, P), jnp.float32)
    if final:
        zbuf_ref[:, 0:P] = jnp.zeros((36, P), jnp.float32)
        zbuf_ref[:, P + Tb:2 * P + Tb] = jnp.zeros((36, P), jnp.float32)

    for b in range(_B):
        if route_vecs is None:
            i1 = idx_ref[blk, 2 * b + 0]
            i2 = idx_ref[blk, 2 * b + 1]
            g1 = wts_ref[blk, 2 * b + 0]
            g2 = wts_ref[blk, 2 * b + 1]
        else:
            (iv1, iv2), (wv1, wv2) = route_vecs[0][0], route_vecs[1][0]
            i1 = jnp.sum(iv1[b:b + 1, 0:1])
            i2 = jnp.sum(iv2[b:b + 1, 0:1])
            g1 = jnp.sum(wv1[b:b + 1, 0:1])
            g2 = jnp.sum(wv2[b:b + 1, 0:1])
        x = x_ref[b]
        xb = x.astype(jnp.bfloat16)
        w1a = w1_ref[pl.ds(i1, 1)][0]
        w1b = w1_ref[pl.ds(i2, 1)][0]
        h1 = jax.nn.gelu(_dotg(w1a, xb))
        h2 = jax.nn.gelu(_dotg(w1b, xb))
        w2a = w2_ref[pl.ds(i1, 1)][0]
        w2b = w2_ref[pl.ds(i2, 1)][0]
        y = x + g1 * _dotg(w2a, h1) + g2 * _dotg(w2b, h2)
        ybuf_ref[:, P:P + Tb] = y

        acc = None
        for dw in (-1, 0, 1):
            part = None
            for dh in (-1, 0, 1):
                off = dh * W + dw
                ys = ybuf_ref[:, P + off:P + off + Tb]
                tap = 3 * (dh + 1) + (dw + 1)
                c = _dotg(wu_ref[tap], ys)
                part = c if part is None else part + c
            if wmask[dw] is not None:
                part = jnp.where(wmask[dw], part, 0.0)
            acc = part if acc is None else acc + part
        y2 = jnp.where(acc >= 0, acc, 0.01 * acc)

        if not final:
            out_ref[b] = y2
            continue

        z = _dotg(wz_ref[:], y2)
        zbuf_ref[:, P:P + Tb] = z
        img = None
        for dw in (-1, 0, 1):
            part = None
            for dh in (-1, 0, 1):
                off = dh * W + dw
                tap = 3 * (dh + 1) + (dw + 1)
                zs = zbuf_ref[4 * tap:4 * tap + 4, P + off:P + off + Tb]
                part = zs if part is None else part + zs
            if wmask[dw] is not None:
                part = jnp.where(wmask[dw], part, 0.0)
            img = part if img is None else img + part
        out_ref[b] = img


def _wz_indices():
    oc = np.zeros((64, 36), np.int32)
    kh = np.zeros((64, 36), np.int32)
    kw = np.zeros((64, 36), np.int32)
    msk = np.zeros((64, 36), np.float32)
    for r1 in (0, 1):
        for r2 in (0, 1):
            for sh in (-1, 0, 1):
                for sw in (-1, 0, 1):
                    for s1 in (0, 1):
                        for s2 in (0, 1):
                            dh = 2 * sh + s1 - r1
                            dw = 2 * sw + s2 - r2
                            if -1 <= dh <= 1 and -1 <= dw <= 1:
                                tap = 3 * (sh + 1) + (sw + 1)
                                col = 4 * tap + r1 * 2 + r2
                                rows = np.arange(s1 * 2 + s2, 64, 4)
                                oc[rows, col] = np.arange(16)
                                kh[rows, col] = dh + 1
                                kw[rows, col] = dw + 1
                                msk[rows, col] = 1.0
    return oc, kh, kw, msk


_WZ_OC, _WZ_KH, _WZ_KW, _WZ_MSK = _wz_indices()


def _build_wz(wc):
    return wc[0][_WZ_OC, _WZ_KH, _WZ_KW] * _WZ_MSK


def _shuffle(y, o):
    n, c4, h, w = y.shape
    y = y.reshape(n, o, 2, 2, h, w)
    y = y.transpose(0, 1, 4, 2, 5, 3)
    return y.reshape(n, o, 2 * h, 2 * w)


def _stage_call(blk, x, idx, wts, p, wz=None, route=None):
    d = _DIMS[blk]
    o = _OUTS[blk]
    H = _HS[blk]
    Tb = H * H
    P = H + 8
    final = wz is not None

    w1 = p['W1'].astype(jnp.bfloat16)
    w2 = p['W2'].astype(jnp.bfloat16)
    wu = p['Wu'].transpose(2, 3, 1, 0).reshape(9, d, 4 * o).astype(jnp.bfloat16)
    smem = pl.BlockSpec(memory_space=pltpu.SMEM)
    vmem = pl.BlockSpec(memory_space=pltpu.VMEM)

    if route is not None:
        txt, wg = route
        args = [txt, wg, x.reshape(_B, d, Tb), w1, w2, wu]
    else:
        args = [idx, wts, x.reshape(_B, d, Tb), w1, w2, wu]
        smems = [smem, smem]
    specs = ([vmem, vmem] if route is not None else [smem, smem]) \
        + [vmem, vmem, vmem, vmem]
    scratch = [pltpu.VMEM((d, Tb + 2 * P), jnp.float32)]
    out_c = 4 if final else 4 * o
    if final:
        args.append(wz.astype(jnp.bfloat16))
        specs.append(vmem)
        scratch.append(pltpu.VMEM((36, Tb + 2 * P), jnp.float32))

    out_shape = jax.ShapeDtypeStruct((_B, out_c, Tb), jnp.float32)
    if route is not None:
        out_shape = (jax.ShapeDtypeStruct((4, _B, 2), jnp.int32),
                     jax.ShapeDtypeStruct((4, _B, 2), jnp.float32),
                     jax.ShapeDtypeStruct((1, 1), jnp.float32),
                     out_shape)
        idxo, wtso, mio, y = pl.pallas_call(
            functools.partial(_stage_body, blk, final),
            out_shape=out_shape,
            in_specs=specs,
            out_specs=(vmem, vmem, vmem, vmem),
            scratch_shapes=scratch,
        )(*args)
        return idxo, wtso, mio, y.reshape(_B, out_c, H, H)
    y = pl.pallas_call(
        functools.partial(_stage_body, blk, final),
        out_shape=out_shape,
        in_specs=specs,
        out_specs=vmem,
        scratch_shapes=scratch,
    )(*args)
    return y.reshape(_B, out_c, H, H)


def kernel(x_encode_0, x_encode_1, x_encode_2, x_encode_3, text_feature, params):
    wg = jnp.concatenate([params['blk%d' % k]['Wg'] for k in range(4)], axis=1)

    skips = [None, x_encode_2, x_encode_1, x_encode_0]
    idx, wts, mi, y = _stage_call(0, x_encode_3, None, None, params['blk0'],
                                  route=(text_feature, wg))
    idx = idx.reshape(4, 2 * _B)
    wts = wts.reshape(4, 2 * _B)
    cur = skips[1] + _shuffle(y, _OUTS[0])
    for k in range(1, 3):
        y = _stage_call(k, cur, idx, wts, params['blk%d' % k])
        cur = skips[k + 1] + _shuffle(y, _OUTS[k])

    wz = _build_wz(params['Wc'])
    img4 = _stage_call(3, cur, idx, wts, params['blk3'], wz=wz)
    img = _shuffle(img4, 1)
    return img, mi.reshape(())
